# baseline (speedup 1.0000x reference)
import functools

import jax
import jax.numpy as jnp
from jax import lax
from jax.experimental import pallas as pl
from jax.experimental.pallas import tpu as pltpu
from jax.experimental.pallas import tpu_sc as plsc

_NC = 2
_NS = 16
_C = 128
_RPB = 16
_NB = 6
_KA = 4


def _linear_body(x_ref, dp_ref, dd_ref, yp_ref, yd_ref):
    x = x_ref[...]
    yp_ref[...] = jnp.dot(x, dp_ref[...], preferred_element_type=jnp.float32)
    yd_ref[...] = jnp.dot(x, dd_ref[...], preferred_element_type=jnp.float32)


def _linear(xp, dp, dd):
    m = xp.shape[0]
    bm = m // 17 if m % 17 == 0 else 512
    spec = pl.BlockSpec((bm, 128), lambda i: (i, 0))
    wspec = pl.BlockSpec((128, 128), lambda i: (0, 0))
    out = jax.ShapeDtypeStruct((m, 128), jnp.float32)
    return pl.pallas_call(
        _linear_body,
        grid=(m // bm,),
        in_specs=[spec, wspec, wspec],
        out_specs=[spec, spec],
        out_shape=[out, out],
    )(xp, dp, dd)


def _spmm_layer(npad, epad):
    ept = epad // _NS
    nbatch = ept // (_C * _RPB)
    crpt = ept // _C
    rpt = npad // _NS
    fb_full = rpt // _C
    fb_rem = rpt - fb_full * _C
    mesh = plsc.VectorSubcoreMesh(core_axis_name="c", subcore_axis_name="s")

    @functools.partial(
        pl.kernel,
        out_type=jax.ShapeDtypeStruct((npad, 32), jnp.float32),
        mesh=mesh,
        compiler_params=pltpu.CompilerParams(use_tc_tiling_on_sc=False, needs_layout_passes=False),
        scratch_types=(
            [pltpu.VMEM_SHARED((npad, 8), jnp.float32)]
            + [pltpu.VMEM_SHARED((npad, 8), jnp.float32)]
            + [pltpu.VMEM((_RPB, _C), jnp.int32)] * 4
            + [pltpu.VMEM((_RPB, _C), jnp.float32)] * 2
            + [pltpu.VMEM((_C, 8), jnp.float32)] * _NB
            + [pltpu.VMEM((_C, 8), jnp.float32)]
            + [pltpu.SemaphoreType.DMA] * 0
            + [pltpu.SemaphoreType.DMA] * (2 + 2 * _NB)
        ),
    )
    def k(yp_hbm, yd_hbm, esrc, edst, evals, bf_hbm, out_hbm,
          ybuf, acc, src0, src1, dst0, dst1, val0, val1, *rest):
        rows = rest[:_NB]
        fillb = rest[_NB]
        semI = rest[_NB + 1:_NB + 3]
        semG = rest[_NB + 3:_NB + 3 + _NB]
        semS = rest[_NB + 3 + _NB:_NB + 3 + 2 * _NB]
        c = lax.axis_index("c")
        s = lax.axis_index("s")
        srcB, dstB, valB = (src0, src1), (dst0, dst1), (val0, val1)
        nbase = s * rpt

        def scat_wait(t):
            pltpu.make_async_copy(rows[t % _NB], acc.at[dstB[0].at[0]],
                                  semS[t % _NB]).wait()

        def edge_pass(p):
            rbase = s * crpt

            def idx_descs(bi, kk):
                ro = rbase + bi * _RPB
                return (
                    pltpu.make_async_copy(esrc.at[p, pl.ds(ro, _RPB)], srcB[kk], semI[kk]),
                    pltpu.make_async_copy(edst.at[p, pl.ds(ro, _RPB)], dstB[kk], semI[kk]),
                    pltpu.make_async_copy(evals.at[p, pl.ds(ro, _RPB)], valB[kk], semI[kk]),
                )

            for d in idx_descs(0, 0):
                d.start()

            @pl.loop(0, nbatch, step=2)
            def _(bi):
                for kk in (0, 1):
                    b = bi + kk
                    for d in idx_descs(b, kk):
                        d.wait()

                    @pl.when(b + 1 < nbatch)
                    def _():
                        for d in idx_descs(b + 1, kk ^ 1):
                            d.start()

                    def gissue(t):
                        return pltpu.async_copy(ybuf.at[srcB[kk].at[t]],
                                                rows[t % _NB], semG[t % _NB])

                    gd = {}
                    sd = {}
                    for t in range(_KA):

                        @pl.when(b > 0)
                        def _(t=t):
                            scat_wait(t)

                        gd[t] = gissue(t)

                    for ch in range(_RPB):
                        gd[ch].wait()
                        t = ch + _KA
                        if t < _RPB:
                            if t < _NB:

                                @pl.when(b > 0)
                                def _(t=t):
                                    scat_wait(t)

                            else:
                                sd[t - _NB].wait()
                            gd[t] = gissue(t)
                        rb = rows[ch % _NB]

                        @pl.loop(0, 8)
                        def _(g, ch=ch, rb=rb):
                            vp = jnp.where(lax.iota(jnp.int32, 16) < 8, 0, 1)
                            colp = lax.iota(jnp.int32, 16) & 7
                            chv = jnp.full((16,), ch, jnp.int32)
                            for j in range(8):
                                idx_e = vp + (g * 16 + 2 * j)
                                vv = plsc.load_gather(valB[kk], [chv, idx_e])
                                xv = plsc.load_gather(rb, [idx_e, colp])
                                plsc.store_scatter(rb, [idx_e, colp], xv * vv)

                        sd[ch] = pltpu.make_async_copy(
                            rows[ch % _NB], acc.at[dstB[kk].at[ch]],
                            semS[ch % _NB])
                        sd[ch].start(add=True)

            for t in range(_RPB - _NB, _RPB):
                scat_wait(t)

        @pl.loop(0, 2)
        def _(q):
            qq = 2 * c + q
            col = 8 * qq
            pltpu.sync_copy(yp_hbm.at[pl.ds(nbase, rpt), pl.ds(col, 8)],
                            ybuf.at[pl.ds(nbase, rpt)])
            for j in range(_C // 16):
                pltpu.sync_copy(bf_hbm.at[qq], fillb.at[pl.ds(j * 16, 16)])
            for j in range(fb_full):
                pltpu.sync_copy(fillb, acc.at[pl.ds(nbase + j * _C, _C)])
            if fb_rem:
                pltpu.sync_copy(fillb.at[pl.ds(0, fb_rem)],
                                acc.at[pl.ds(nbase + fb_full * _C, fb_rem)])
            plsc.subcore_barrier()

            @pl.loop(0, 2)
            def _(p, col=col):

                @pl.when(p == 1)
                def _():
                    plsc.subcore_barrier()
                    pltpu.sync_copy(yd_hbm.at[pl.ds(nbase, rpt), pl.ds(col, 8)],
                                    ybuf.at[pl.ds(nbase, rpt)])
                    plsc.subcore_barrier()

                edge_pass(p)

            plsc.subcore_barrier()
            pltpu.sync_copy(acc.at[pl.ds(nbase, rpt)],
                            out_hbm.at[pl.ds(nbase, rpt), pl.ds(col, 8)])
            plsc.subcore_barrier()

    return k


def _gather_out(npad, u_count, batch):
    per_w = batch // (_NC * _NS)
    mesh = plsc.VectorSubcoreMesh(core_axis_name="c", subcore_axis_name="s")
    out = jax.ShapeDtypeStruct((batch, 32), jnp.float32)

    @functools.partial(
        pl.kernel,
        out_type=(out, out),
        mesh=mesh,
        compiler_params=pltpu.CompilerParams(use_tc_tiling_on_sc=False, needs_layout_passes=False),
        scratch_types=[
            pltpu.VMEM((_C,), jnp.int32),
            pltpu.VMEM((_C, 32), jnp.float32),
            pltpu.SemaphoreType.DMA,
        ],
    )
    def k(x_hbm, u_hbm, i_hbm, uo_hbm, io_hbm, idxb, rowsb, sem):
        c = lax.axis_index("c")
        s = lax.axis_index("s")
        w = c * _NS + s
        for idx_hbm, off, out_hbm in ((u_hbm, 0, uo_hbm),
                                      (i_hbm, u_count, io_hbm)):
            for ch in range(per_w // _C):
                base = w * per_w + ch * _C
                pltpu.sync_copy(idx_hbm.at[pl.ds(base, _C)], idxb)
                if off:
                    for g in range(_C // 16):
                        sl = pl.ds(g * 16, 16)
                        idxb[sl] = idxb[sl] + off
                pltpu.async_copy(x_hbm.at[idxb], rowsb, sem).wait()
                pltpu.sync_copy(rowsb, out_hbm.at[pl.ds(base, _C)])

    return k


def kernel(users, items, user_emb, item_emb, adj_src, adj_dst, adj_val,
           hp_src, hp_dst, hp_val, W, b):
    u_count, e = user_emb.shape
    n = u_count + item_emb.shape[0]
    npad = ((n + 2047) // 2048) * 2048
    nlayers = W.shape[0]
    batch = users.shape[0]

    ego = jnp.concatenate([user_emb, item_emb], axis=0)
    ego = jnp.pad(ego, ((0, npad - n), (0, 0)))
    xp = ego.reshape(npad // 4, 128)

    nnz = adj_src.shape[0]
    quant = _NS * _C * _RPB * 2
    epad = ((nnz + quant - 1) // quant) * quant
    pad = epad - nnz
    e_src = jnp.stack([jnp.pad(adj_src, (0, pad)).reshape(-1, _C),
                       jnp.pad(hp_src, (0, pad)).reshape(-1, _C)])
    e_dst = jnp.stack([jnp.pad(adj_dst, (0, pad)).reshape(-1, _C),
                       jnp.pad(hp_dst, (0, pad)).reshape(-1, _C)])
    e_val = jnp.stack([jnp.pad(adj_val, (0, pad)).reshape(-1, _C),
                       jnp.pad(hp_val, (0, pad)).reshape(-1, _C)])

    eye4 = jnp.eye(4, dtype=jnp.float32)
    spmm = _spmm_layer(npad, epad)
    for k in range(nlayers):
        dp = jnp.kron(eye4, W[k, :e, :])
        dd = jnp.kron(eye4, W[k, e:, :])
        bf = jnp.tile(b[k].reshape(4, 1, 8), (1, 16, 1))
        ypp, ydp = _linear(xp, dp, dd)
        x = spmm(ypp.reshape(npad, 32), ydp.reshape(npad, 32),
                 e_src, e_dst, e_val, bf)
        xp = x.reshape(npad // 4, 128)

    uo, io = _gather_out(npad, u_count, batch)(x, users, items)
    return (uo, io)


# speedup vs baseline: 1.7243x; 1.7243x over previous
import functools

import jax
import jax.numpy as jnp
from jax import lax
from jax.experimental import pallas as pl
from jax.experimental.pallas import tpu as pltpu
from jax.experimental.pallas import tpu_sc as plsc

_NC = 2
_NS = 16
_C = 128
_RPB = 16
_NB = 6
_KA = 4


def _linear_body(xlo_ref, xhi_ref, dp_ref, dd_ref, yp_ref, yd_ref):
    xlo = xlo_ref[...]
    xhi = xhi_ref[...]
    yp_ref[...] = (
        jnp.dot(xlo, dp_ref[0, 0], preferred_element_type=jnp.float32)
        + jnp.dot(xhi, dp_ref[0, 1], preferred_element_type=jnp.float32))
    yd_ref[...] = (
        jnp.dot(xlo, dd_ref[0, 0], preferred_element_type=jnp.float32)
        + jnp.dot(xhi, dd_ref[0, 1], preferred_element_type=jnp.float32))


def _linear(xp, dp, dd):
    m = xp.shape[0]
    half = m // 2
    grid_i = 17 if half % 17 == 0 else 8
    bm = half // grid_i
    out = jax.ShapeDtypeStruct((m, 128), jnp.float32)
    xlo_spec = pl.BlockSpec((bm, 128), lambda i, h: (i, 0))
    xhi_spec = pl.BlockSpec((bm, 128), lambda i, h, gi=grid_i: (gi + i, 0))
    d_spec = pl.BlockSpec((1, 2, 128, 128), lambda i, h: (h, 0, 0, 0))
    o_spec = pl.BlockSpec((bm, 128), lambda i, h, gi=grid_i: (h * gi + i, 0))
    return pl.pallas_call(
        _linear_body,
        grid=(grid_i, 2),
        in_specs=[xlo_spec, xhi_spec, d_spec, d_spec],
        out_specs=[o_spec, o_spec],
        out_shape=[out, out],
    )(xp, xp, dp, dd)


def _spmm_layer(npad, epad):
    ept = epad // _NS
    nbatch = ept // (_C * _RPB)
    crpt = ept // _C
    rpt = npad // _NS
    fb_full = rpt // _C
    fb_rem = rpt - fb_full * _C
    mesh = plsc.VectorSubcoreMesh(core_axis_name="c", subcore_axis_name="s")

    @functools.partial(
        pl.kernel,
        out_type=jax.ShapeDtypeStruct((2 * npad, 16), jnp.float32),
        mesh=mesh,
        compiler_params=pltpu.CompilerParams(use_tc_tiling_on_sc=False),
        scratch_types=(
            [pltpu.VMEM_SHARED((npad, 16), jnp.float32)]
            + [pltpu.VMEM((_RPB, _C), jnp.int32)] * 4
            + [pltpu.VMEM((_RPB, _C), jnp.float32)] * 2
            + [pltpu.VMEM((_C, 16), jnp.float32)] * _NB
            + [pltpu.VMEM((_C, 16), jnp.float32)]
            + [pltpu.SemaphoreType.DMA] * (2 + 2 * _NB)
        ),
    )
    def k(yp_hbm, yd_hbm, asrc, adst, aval, hsrc, hdst, hval, b_hbm, out_hbm,
          acc, src0, src1, dst0, dst1, val0, val1, *rest):
        rows = rest[:_NB]
        fillb = rest[_NB]
        semI = rest[_NB + 1:_NB + 3]
        semG = rest[_NB + 3:_NB + 3 + _NB]
        semS = rest[_NB + 3 + _NB:_NB + 3 + 2 * _NB]
        c = lax.axis_index("c")
        s = lax.axis_index("s")
        cn = c * npad
        srcB, dstB, valB = (src0, src1), (dst0, dst1), (val0, val1)

        pltpu.sync_copy(b_hbm.at[pl.ds(c * 16, 16)], fillb.at[0])
        bv = fillb[0, :]
        for r in range(1, _C):
            fillb[r, :] = bv
        ibase = s * rpt
        for j in range(fb_full):
            pltpu.sync_copy(fillb, acc.at[pl.ds(ibase + j * _C, _C)])
        if fb_rem:
            pltpu.sync_copy(fillb.at[pl.ds(0, fb_rem)],
                            acc.at[pl.ds(ibase + fb_full * _C, fb_rem)])
        plsc.subcore_barrier()

        def scat_wait(t):
            pltpu.make_async_copy(rows[t % _NB], acc.at[dstB[0].at[0]],
                                  semS[t % _NB]).wait()

        def edge_pass(src2, dst2, val2, y_hbm):
            rbase = s * crpt

            def idx_descs(bi, kk):
                ro = rbase + bi * _RPB
                return (
                    pltpu.make_async_copy(src2.at[pl.ds(ro, _RPB)], srcB[kk], semI[kk]),
                    pltpu.make_async_copy(dst2.at[pl.ds(ro, _RPB)], dstB[kk], semI[kk]),
                    pltpu.make_async_copy(val2.at[pl.ds(ro, _RPB)], valB[kk], semI[kk]),
                )

            for d in idx_descs(0, 0):
                d.start()

            @pl.loop(0, nbatch, step=2)
            def _(bi):
                for kk in (0, 1):
                    b = bi + kk
                    for d in idx_descs(b, kk):
                        d.wait()

                    @pl.when(b + 1 < nbatch)
                    def _():
                        for d in idx_descs(b + 1, kk ^ 1):
                            d.start()

                    @pl.loop(0, _RPB)
                    def _(r):
                        for g in range(_C // 16):
                            sl = pl.ds(g * 16, 16)
                            srcB[kk][r, sl] = srcB[kk][r, sl] + cn

                    def gissue(t):
                        return pltpu.async_copy(y_hbm.at[srcB[kk].at[t]],
                                                rows[t % _NB], semG[t % _NB])

                    gd = {}
                    sd = {}
                    for t in range(_KA):

                        @pl.when(b > 0)
                        def _(t=t):
                            scat_wait(t)

                        gd[t] = gissue(t)

                    for ch in range(_RPB):
                        gd[ch].wait()
                        t = ch + _KA
                        if t < _RPB:
                            if t < _NB:

                                @pl.when(b > 0)
                                def _(t=t):
                                    scat_wait(t)

                            else:
                                sd[t - _NB].wait()
                            gd[t] = gissue(t)
                        rb = rows[ch % _NB]

                        @pl.loop(0, _C // 16)
                        def _(g, ch=ch, rb=rb):
                            gb = g * 16
                            vv = valB[kk][ch, pl.ds(gb, 16)]
                            for j in range(16):
                                rb[gb + j, :] = rb[gb + j, :] * vv[j]

                        sd[ch] = pltpu.make_async_copy(
                            rows[ch % _NB], acc.at[dstB[kk].at[ch]],
                            semS[ch % _NB])
                        sd[ch].start(add=True)

            for t in range(_RPB - _NB, _RPB):
                scat_wait(t)

        edge_pass(asrc, adst, aval, yp_hbm)
        edge_pass(hsrc, hdst, hval, yd_hbm)
        plsc.subcore_barrier()

        pltpu.sync_copy(acc.at[pl.ds(ibase, rpt)],
                        out_hbm.at[pl.ds(cn + ibase, rpt)])

    return k


def _gather_out(npad, u_count, batch):
    per_tile = batch // _NS
    mesh = plsc.VectorSubcoreMesh(core_axis_name="c", subcore_axis_name="s")
    out = jax.ShapeDtypeStruct((2 * batch, 16), jnp.float32)

    @functools.partial(
        pl.kernel,
        out_type=(out, out),
        mesh=mesh,
        compiler_params=pltpu.CompilerParams(use_tc_tiling_on_sc=False),
        scratch_types=[
            pltpu.VMEM((_C,), jnp.int32),
            pltpu.VMEM((_C, 16), jnp.float32),
            pltpu.SemaphoreType.DMA,
        ],
    )
    def k(x_hbm, u_hbm, i_hbm, uo_hbm, io_hbm, idxb, rowsb, sem):
        c = lax.axis_index("c")
        s = lax.axis_index("s")
        for idx_hbm, off, out_hbm in ((u_hbm, c * npad, uo_hbm),
                                      (i_hbm, c * npad + u_count, io_hbm)):
            for ch in range(per_tile // _C):
                base = s * per_tile + ch * _C
                pltpu.sync_copy(idx_hbm.at[pl.ds(base, _C)], idxb)
                for g in range(_C // 16):
                    sl = pl.ds(g * 16, 16)
                    idxb[sl] = idxb[sl] + off
                pltpu.async_copy(x_hbm.at[idxb], rowsb, sem).wait()
                pltpu.sync_copy(rowsb, out_hbm.at[pl.ds(c * batch + base, _C)])

    return k


def kernel(users, items, user_emb, item_emb, adj_src, adj_dst, adj_val,
           hp_src, hp_dst, hp_val, W, b):
    u_count, e = user_emb.shape
    eh = e // 2
    n = u_count + item_emb.shape[0]
    npad = ((n + 135) // 136) * 136
    nlayers = W.shape[0]
    batch = users.shape[0]

    ego = jnp.concatenate([user_emb, item_emb], axis=0)
    ego = jnp.pad(ego, ((0, npad - n), (0, 0)))
    xp = jnp.concatenate([ego[:, :eh].reshape(-1, 128),
                          ego[:, eh:].reshape(-1, 128)], axis=0)

    nnz = adj_src.shape[0]
    quant = _NS * _C * _RPB * 2
    epad = ((nnz + quant - 1) // quant) * quant
    pad = epad - nnz
    a_src = jnp.pad(adj_src, (0, pad)).reshape(-1, _C)
    a_dst = jnp.pad(adj_dst, (0, pad)).reshape(-1, _C)
    a_val = jnp.pad(adj_val, (0, pad)).reshape(-1, _C)
    h_src = jnp.pad(hp_src, (0, pad)).reshape(-1, _C)
    h_dst = jnp.pad(hp_dst, (0, pad)).reshape(-1, _C)
    h_val = jnp.pad(hp_val, (0, pad)).reshape(-1, _C)

    eye8 = jnp.eye(8, dtype=jnp.float32)
    spmm = _spmm_layer(npad, epad)
    for k in range(nlayers):
        wp, wd = W[k, :e, :], W[k, e:, :]
        dp = jnp.stack([
            jnp.stack([jnp.kron(eye8, wp[:eh, h * eh:(h + 1) * eh]),
                       jnp.kron(eye8, wp[eh:, h * eh:(h + 1) * eh])])
            for h in range(2)])
        dd = jnp.stack([
            jnp.stack([jnp.kron(eye8, wd[:eh, h * eh:(h + 1) * eh]),
                       jnp.kron(eye8, wd[eh:, h * eh:(h + 1) * eh])])
            for h in range(2)])
        ypp, ydp = _linear(xp, dp, dd)
        x_flat = spmm(ypp.reshape(2 * npad, 16), ydp.reshape(2 * npad, 16),
                      a_src, a_dst, a_val, h_src, h_dst, h_val, b[k])
        xp = x_flat.reshape(2 * npad // 8, 128)

    uo, io = _gather_out(npad, u_count, batch)(x_flat, users, items)
    u2 = uo.reshape(2, batch, 16)
    i2 = io.reshape(2, batch, 16)
    user_embeddings = jnp.concatenate([u2[0], u2[1]], axis=1)
    item_embeddings = jnp.concatenate([i2[0], i2[1]], axis=1)
    return (user_embeddings, item_embeddings)
